# baseline (speedup 1.0000x reference)
import dataclasses
import functools

import jax
import jax.numpy as jnp
from jax import lax
from jax.experimental import pallas as pl
from jax.experimental.pallas import tpu as pltpu
from jax.experimental.pallas import tpu_sc as plsc

_OUT_H, _OUT_W = 7, 7
_SCALE = 0.25
_GRID = 2
_NC, _NS, _L = 2, 16, 16
_NW = _NC * _NS


def _tap_tables(rois, n, h, w):
    r = rois.shape[0]
    ph, pw, g = _OUT_H, _OUT_W, _GRID
    offset = 0.5
    batch_idx = rois[:, 0].astype(jnp.int32)
    ctr_x = rois[:, 1] * _SCALE - offset
    ctr_y = rois[:, 2] * _SCALE - offset
    roi_w = rois[:, 3] * _SCALE
    roi_h = rois[:, 4] * _SCALE
    theta = rois[:, 5] * jnp.pi / 180.0
    cos_t, sin_t = jnp.cos(theta), jnp.sin(theta)
    bin_h = roi_h / ph
    bin_w = roi_w / pw
    iy = (jnp.arange(g, dtype=jnp.float32) + 0.5) / g
    py = jnp.arange(ph, dtype=jnp.float32)
    yy = (-roi_h[:, None, None] / 2.0) + (py[None, :, None] + iy[None, None, :]) * bin_h[:, None, None]
    xx = (-roi_w[:, None, None] / 2.0) + (py[None, :, None] + iy[None, None, :]) * bin_w[:, None, None]
    yy_b = yy[:, :, None, :, None]
    xx_b = xx[:, None, :, None, :]
    c5 = cos_t[:, None, None, None, None]
    s5 = sin_t[:, None, None, None, None]
    y = yy_b * c5 - xx_b * s5 + ctr_y[:, None, None, None, None]
    x = yy_b * s5 + xx_b * c5 + ctr_x[:, None, None, None, None]
    s = ph * pw * g * g
    y = y.reshape(r, s)
    x = x.reshape(r, s)
    valid = (y > -1.0) & (y < h) & (x > -1.0) & (x < w)
    y = jnp.maximum(y, 0.0)
    x = jnp.maximum(x, 0.0)
    yl = jnp.floor(y).astype(jnp.int32)
    xl = jnp.floor(x).astype(jnp.int32)
    y = jnp.where(yl >= h - 1, jnp.float32(h - 1), y)
    x = jnp.where(xl >= w - 1, jnp.float32(w - 1), x)
    yl = jnp.minimum(yl, h - 1)
    xl = jnp.minimum(xl, w - 1)
    yh = jnp.minimum(yl + 1, h - 1)
    xh = jnp.minimum(xl + 1, w - 1)
    ly = y - yl.astype(jnp.float32)
    lx = x - xl.astype(jnp.float32)
    hy = 1.0 - ly
    hx = 1.0 - lx
    scale = jnp.where(valid, 0.25, 0.0)
    base = batch_idx[:, None] * (h * w)
    w00, w01 = hy * hx, hy * lx
    w10, w11 = ly * hx, ly * lx
    xfold = xh == xl
    w00 = jnp.where(xfold, w00 + w01, w00)
    w01 = jnp.where(xfold, 0.0, w01)
    w10 = jnp.where(xfold, w10 + w11, w10)
    w11 = jnp.where(xfold, 0.0, w11)
    yfold = yh == yl
    w00 = jnp.where(yfold, w00 + w10, w00)
    w10 = jnp.where(yfold, 0.0, w10)
    w01 = jnp.where(yfold, w01 + w11, w01)
    w11 = jnp.where(yfold, 0.0, w11)
    idx = base + yl * w + xl
    return (
        idx.reshape(-1),
        tuple((wc * scale).reshape(-1) for wc in (w00, w01, w10, w11)),
    )


def _sc_call(n_bins, cb, chunk):
    per_w = n_bins // _NW
    steps = per_w // chunk
    mesh = plsc.VectorSubcoreMesh(
        core_axis_name="c", subcore_axis_name="s", num_cores=_NC, num_subcores=_NS
    )
    cp = pltpu.CompilerParams()
    if "needs_layout_passes" in pltpu.CompilerParams.__dataclass_fields__:
        cp = dataclasses.replace(cp, needs_layout_passes=False)

    @functools.partial(
        pl.kernel,
        compiler_params=cp,
        out_type=jax.ShapeDtypeStruct((n_bins, cb), jnp.float32),
        mesh=mesh,
        scratch_types=[
            pltpu.VMEM((chunk * 4,), jnp.int32),
            pltpu.VMEM((chunk * 4,), jnp.int32),
            [pltpu.VMEM((per_w * 4,), jnp.float32) for _ in range(4)],
            pltpu.VMEM((2, chunk * 4, 2 * cb), jnp.int32),
            pltpu.VMEM((2, chunk, cb), jnp.float32),
            pltpu.SemaphoreType.DMA,
            pltpu.SemaphoreType.DMA,
            pltpu.SemaphoreType.DMA,
            pltpu.SemaphoreType.DMA,
            pltpu.SemaphoreType.DMA,
            pltpu.SemaphoreType.DMA,
        ],
    )
    def kern(feat_hbm, idx_hbm, w00_hbm, w01_hbm, w10_hbm, w11_hbm, out_hbm,
             idx_a, idx_b, w_vs, rows_v, out_v,
             gsem0, gsem1, osem0, osem1, isema, isemb):
        wid = lax.axis_index("s") * _NC + lax.axis_index("c")
        base_bin = wid * per_w
        for w_hbm, w_v in zip((w00_hbm, w01_hbm, w10_hbm, w11_hbm), w_vs):
            pltpu.sync_copy(w_hbm.at[pl.ds(base_bin * 4, per_w * 4)], w_v)

        def idx_desc(s, buf, sem):
            t0 = (base_bin + s * chunk) * 4
            return pltpu.make_async_copy(
                idx_hbm.at[pl.ds(t0, chunk * 4)], buf, sem
            )

        def gather_desc(buf, p, sem):
            return pltpu.make_async_copy(feat_hbm.at[buf], rows_v.at[p], sem)

        def out_desc(s, p, sem):
            return pltpu.make_async_copy(
                out_v.at[p], out_hbm.at[pl.ds(base_bin + s * chunk, chunk)], sem
            )

        def compute(s, p):
            rbuf = rows_v.at[p]
            obuf = out_v.at[p]
            nblk = cb // (2 * _L)

            @pl.loop(0, chunk)
            def _(b):
                tb = s * (chunk * 4) + b * 4

                def sample(smp, acc):
                    rrow = b * 4 + smp
                    acc = list(acc)
                    for corner in range(4):
                        wv = plsc.load_gather(
                            w_vs[corner],
                            [jnp.full((_L,), tb + smp, jnp.int32)],
                        )
                        for j in range(nblk):
                            v = plsc.bitcast(
                                rbuf[rrow, pl.ds(corner * cb // 2 + j * _L, _L)],
                                jnp.bfloat16,
                            )
                            lo, hi = plsc.unpack(
                                v, format=plsc.PackFormat.INTERLEAVED
                            )
                            acc[2 * j] = acc[2 * j] + wv * lo
                            acc[2 * j + 1] = acc[2 * j + 1] + wv * hi
                    return tuple(acc)

                acc0 = tuple(
                    jnp.zeros((_L,), jnp.float32) for _ in range(2 * nblk)
                )
                acc = lax.fori_loop(0, 4, sample, acc0)
                for j in range(nblk):
                    obuf[b, pl.ds(j * _L, _L)] = acc[2 * j]
                    obuf[b, pl.ds(cb // 2 + j * _L, _L)] = acc[2 * j + 1]

        idx_desc(0, idx_a, isema).start()
        idx_desc(0, idx_a, isema).wait()
        gather_desc(idx_a, 0, gsem0).start()
        idx_desc(1, idx_b, isemb).start()
        half = steps // 2

        @pl.loop(0, half)
        def _(i):
            s0 = 2 * i
            s1 = s0 + 1
            idx_desc(s1, idx_b, isemb).wait()
            gather_desc(idx_b, 1, gsem1).start()
            gather_desc(idx_a, 0, gsem0).wait()

            @pl.when(i < half - 1)
            def _():
                idx_desc(s0 + 2, idx_a, isema).start()

            @pl.when(i > 0)
            def _():
                out_desc(s0 - 2, 0, osem0).wait()

            compute(s0, 0)
            out_desc(s0, 0, osem0).start()

            @pl.when(i < half - 1)
            def _():
                idx_desc(s0 + 2, idx_a, isema).wait()
                gather_desc(idx_a, 0, gsem0).start()

            gather_desc(idx_b, 1, gsem1).wait()

            @pl.when(i < half - 1)
            def _():
                idx_desc(s1 + 2, idx_b, isemb).start()

            @pl.when(i > 0)
            def _():
                out_desc(s1 - 2, 1, osem1).wait()

            compute(s1, 1)
            out_desc(s1, 1, osem1).start()

        out_desc(steps - 2, 0, osem0).wait()
        out_desc(steps - 1, 1, osem1).wait()

    return kern


def _build_quad_tc(input):
    n, c, h, w = input.shape
    hw = h * w
    jb = c // 2

    def body(lo_ref, hi_ref, o_ref):
        lo = jnp.transpose(lo_ref[0].reshape(jb, hw), (1, 0))
        hi = jnp.transpose(hi_ref[0].reshape(jb, hw), (1, 0))
        lo = lax.bitcast_convert_type(lo.astype(jnp.bfloat16), jnp.uint16)
        hi = lax.bitcast_convert_type(hi.astype(jnp.bfloat16), jnp.uint16)
        words = lo.astype(jnp.uint32) | (hi.astype(jnp.uint32) << 16)
        o_ref[...] = lax.bitcast_convert_type(words, jnp.int32)

    words = pl.pallas_call(
        body,
        grid=(n,),
        in_specs=[
            pl.BlockSpec((1, jb, h, w), lambda i: (i, 0, 0, 0)),
            pl.BlockSpec((1, jb, h, w), lambda i: (i, 1, 0, 0)),
        ],
        out_specs=pl.BlockSpec((hw, jb), lambda i: (i, 0)),
        out_shape=jax.ShapeDtypeStruct((n * hw, c // 2), jnp.int32),
    )(input, input)
    wp = jnp.pad(words, ((0, w + 2), (0, 0)))
    nhw = n * hw
    return jnp.concatenate(
        [wp[:nhw], wp[1 : nhw + 1], wp[w : nhw + w], wp[w + 1 : nhw + w + 1]],
        axis=1,
    )


def _post_tc(sc_out, r, c):
    rb = 8

    def body(x_ref, o_ref):
        x = x_ref[...].reshape(rb, 49, c)
        xt = jnp.transpose(x, (0, 2, 1))
        for py in range(_OUT_H):
            o_ref[:, :, py, :] = xt[:, :, py * _OUT_W : (py + 1) * _OUT_W]

    return pl.pallas_call(
        body,
        grid=(r // rb,),
        in_specs=[pl.BlockSpec((rb * 49, c), lambda i: (i, 0))],
        out_specs=pl.BlockSpec((rb, c, _OUT_H, _OUT_W), lambda i: (i, 0, 0, 0)),
        out_shape=jax.ShapeDtypeStruct((r, c, _OUT_H, _OUT_W), jnp.float32),
    )(sc_out)


def kernel(input, rois):
    n, c, h, w = input.shape
    r = rois.shape[0]
    quad = _build_quad_tc(input)
    idx, wcs = _tap_tables(rois, n, h, w)
    rp = 1024
    n_bins = rp * 49
    npad = n_bins * 4 - r * 196
    idx = jnp.pad(idx, (0, npad))
    wcs = [jnp.pad(wc, (0, npad)) for wc in wcs]
    out = _sc_call(n_bins, c, 8)(quad, idx, *wcs)
    return _post_tc(out, r, c)


# speedup vs baseline: 2.9307x; 2.9307x over previous
import dataclasses
import functools

import jax
import jax.numpy as jnp
from jax import lax
from jax.experimental import pallas as pl
from jax.experimental.pallas import tpu as pltpu
from jax.experimental.pallas import tpu_sc as plsc

_OUT_H, _OUT_W = 7, 7
_SCALE = 0.25
_GRID = 2
_NC, _NS, _L = 2, 16, 16
_NW = _NC * _NS


def _tap_tables(rois, n, h, w):
    r = rois.shape[0]
    ph, pw, g = _OUT_H, _OUT_W, _GRID
    offset = 0.5
    batch_idx = rois[:, 0].astype(jnp.int32)
    ctr_x = rois[:, 1] * _SCALE - offset
    ctr_y = rois[:, 2] * _SCALE - offset
    roi_w = rois[:, 3] * _SCALE
    roi_h = rois[:, 4] * _SCALE
    theta = rois[:, 5] * jnp.pi / 180.0
    cos_t, sin_t = jnp.cos(theta), jnp.sin(theta)
    bin_h = roi_h / ph
    bin_w = roi_w / pw
    iy = (jnp.arange(g, dtype=jnp.float32) + 0.5) / g
    py = jnp.arange(ph, dtype=jnp.float32)
    yy = (-roi_h[:, None, None] / 2.0) + (py[None, :, None] + iy[None, None, :]) * bin_h[:, None, None]
    xx = (-roi_w[:, None, None] / 2.0) + (py[None, :, None] + iy[None, None, :]) * bin_w[:, None, None]
    yy_b = yy[:, :, None, :, None]
    xx_b = xx[:, None, :, None, :]
    c5 = cos_t[:, None, None, None, None]
    s5 = sin_t[:, None, None, None, None]
    y = yy_b * c5 - xx_b * s5 + ctr_y[:, None, None, None, None]
    x = yy_b * s5 + xx_b * c5 + ctr_x[:, None, None, None, None]
    s = ph * pw * g * g
    y = y.reshape(r, s)
    x = x.reshape(r, s)
    valid = (y > -1.0) & (y < h) & (x > -1.0) & (x < w)
    y = jnp.maximum(y, 0.0)
    x = jnp.maximum(x, 0.0)
    yl = jnp.floor(y).astype(jnp.int32)
    xl = jnp.floor(x).astype(jnp.int32)
    y = jnp.where(yl >= h - 1, jnp.float32(h - 1), y)
    x = jnp.where(xl >= w - 1, jnp.float32(w - 1), x)
    yl = jnp.minimum(yl, h - 1)
    xl = jnp.minimum(xl, w - 1)
    yh = jnp.minimum(yl + 1, h - 1)
    xh = jnp.minimum(xl + 1, w - 1)
    ly = y - yl.astype(jnp.float32)
    lx = x - xl.astype(jnp.float32)
    hy = 1.0 - ly
    hx = 1.0 - lx
    scale = jnp.where(valid, 0.25, 0.0)
    base = batch_idx[:, None] * (h * w)
    w00, w01 = hy * hx, hy * lx
    w10, w11 = ly * hx, ly * lx
    xfold = xh == xl
    w00 = jnp.where(xfold, w00 + w01, w00)
    w01 = jnp.where(xfold, 0.0, w01)
    w10 = jnp.where(xfold, w10 + w11, w10)
    w11 = jnp.where(xfold, 0.0, w11)
    yfold = yh == yl
    w00 = jnp.where(yfold, w00 + w10, w00)
    w10 = jnp.where(yfold, 0.0, w10)
    w01 = jnp.where(yfold, w01 + w11, w01)
    w11 = jnp.where(yfold, 0.0, w11)
    idx = base + yl * w + xl
    wgt = jnp.stack([w00, w01, w10, w11], axis=2) * scale[:, :, None]
    return idx.reshape(r * 49, 4), wgt.reshape(r * 49, 16)


def _sc_call(n_bins, cb, chunk):
    per_w = n_bins // _NW
    steps = per_w // chunk
    mesh = plsc.VectorSubcoreMesh(
        core_axis_name="c", subcore_axis_name="s", num_cores=_NC, num_subcores=_NS
    )
    cp = pltpu.CompilerParams()
    if "needs_layout_passes" in pltpu.CompilerParams.__dataclass_fields__:
        cp = dataclasses.replace(cp, needs_layout_passes=False)

    @functools.partial(
        pl.kernel,
        compiler_params=cp,
        out_type=jax.ShapeDtypeStruct((n_bins, cb), jnp.float32),
        mesh=mesh,
        scratch_types=[
            pltpu.VMEM((chunk * 4,), jnp.int32),
            pltpu.VMEM((chunk * 4,), jnp.int32),
            pltpu.VMEM((per_w * 16,), jnp.float32),
            pltpu.VMEM((2, chunk * 4, 2 * cb), jnp.int32),
            pltpu.VMEM((2, chunk, cb), jnp.float32),
            pltpu.SemaphoreType.DMA,
            pltpu.SemaphoreType.DMA,
            pltpu.SemaphoreType.DMA,
            pltpu.SemaphoreType.DMA,
            pltpu.SemaphoreType.DMA,
            pltpu.SemaphoreType.DMA,
        ],
    )
    def kern(feat_hbm, idx_hbm, w_hbm, out_hbm, idx_a, idx_b, w_v, rows_v,
             out_v, gsem0, gsem1, osem0, osem1, isema, isemb):
        wid = lax.axis_index("s") * _NC + lax.axis_index("c")
        base_bin = wid * per_w
        pltpu.sync_copy(w_hbm.at[pl.ds(base_bin * 16, per_w * 16)], w_v)

        def idx_desc(s, buf, sem):
            t0 = (base_bin + s * chunk) * 4
            return pltpu.make_async_copy(
                idx_hbm.at[pl.ds(t0, chunk * 4)], buf, sem
            )

        def gather_desc(buf, p, sem):
            return pltpu.make_async_copy(feat_hbm.at[buf], rows_v.at[p], sem)

        def out_desc(s, p, sem):
            return pltpu.make_async_copy(
                out_v.at[p], out_hbm.at[pl.ds(base_bin + s * chunk, chunk)], sem
            )

        def compute(s, p):
            rbuf = rows_v.at[p]
            obuf = out_v.at[p]
            nblk = cb // (2 * _L)

            @pl.loop(0, chunk)
            def _(b):
                tb = s * (chunk * 16) + b * 16

                def sample(smp, acc):
                    rrow = b * 4 + smp
                    acc = list(acc)
                    for corner in range(4):
                        wv = plsc.load_gather(
                            w_v,
                            [jnp.full((_L,), tb + smp * 4 + corner, jnp.int32)],
                        )
                        for j in range(nblk):
                            v = plsc.bitcast(
                                rbuf[rrow, pl.ds(corner * cb // 2 + j * _L, _L)],
                                jnp.bfloat16,
                            )
                            lo, hi = plsc.unpack(
                                v, format=plsc.PackFormat.INTERLEAVED
                            )
                            acc[2 * j] = acc[2 * j] + wv * lo
                            acc[2 * j + 1] = acc[2 * j + 1] + wv * hi
                    return tuple(acc)

                acc0 = tuple(
                    jnp.zeros((_L,), jnp.float32) for _ in range(2 * nblk)
                )
                acc = lax.fori_loop(0, 4, sample, acc0)
                for j in range(nblk):
                    obuf[b, pl.ds(j * _L, _L)] = acc[2 * j]
                    obuf[b, pl.ds(cb // 2 + j * _L, _L)] = acc[2 * j + 1]

        idx_desc(0, idx_a, isema).start()
        idx_desc(0, idx_a, isema).wait()
        gather_desc(idx_a, 0, gsem0).start()
        idx_desc(1, idx_b, isemb).start()
        half = steps // 2

        @pl.loop(0, half)
        def _(i):
            s0 = 2 * i
            s1 = s0 + 1
            idx_desc(s1, idx_b, isemb).wait()
            gather_desc(idx_b, 1, gsem1).start()
            gather_desc(idx_a, 0, gsem0).wait()

            @pl.when(i < half - 1)
            def _():
                idx_desc(s0 + 2, idx_a, isema).start()

            @pl.when(i > 0)
            def _():
                out_desc(s0 - 2, 0, osem0).wait()

            compute(s0, 0)
            out_desc(s0, 0, osem0).start()

            @pl.when(i < half - 1)
            def _():
                idx_desc(s0 + 2, idx_a, isema).wait()
                gather_desc(idx_a, 0, gsem0).start()

            gather_desc(idx_b, 1, gsem1).wait()

            @pl.when(i < half - 1)
            def _():
                idx_desc(s1 + 2, idx_b, isemb).start()

            @pl.when(i > 0)
            def _():
                out_desc(s1 - 2, 1, osem1).wait()

            compute(s1, 1)
            out_desc(s1, 1, osem1).start()

        out_desc(steps - 2, 0, osem0).wait()
        out_desc(steps - 1, 1, osem1).wait()

    return kern


def _build_quad_tc(input):
    n, c, h, w = input.shape
    hw = h * w
    jb = c // 2

    def body(lo_ref, hi_ref, o_ref):
        lo = jnp.transpose(lo_ref[0].reshape(jb, hw), (1, 0))
        hi = jnp.transpose(hi_ref[0].reshape(jb, hw), (1, 0))
        lo = lax.bitcast_convert_type(lo.astype(jnp.bfloat16), jnp.uint16)
        hi = lax.bitcast_convert_type(hi.astype(jnp.bfloat16), jnp.uint16)
        words = lo.astype(jnp.uint32) | (hi.astype(jnp.uint32) << 16)
        o_ref[...] = lax.bitcast_convert_type(words, jnp.int32)

    words = pl.pallas_call(
        body,
        grid=(n,),
        in_specs=[
            pl.BlockSpec((1, jb, h, w), lambda i: (i, 0, 0, 0)),
            pl.BlockSpec((1, jb, h, w), lambda i: (i, 1, 0, 0)),
        ],
        out_specs=pl.BlockSpec((hw, jb), lambda i: (i, 0)),
        out_shape=jax.ShapeDtypeStruct((n * hw, c // 2), jnp.int32),
    )(input, input)
    wp = jnp.pad(words, ((0, w + 2), (0, 0)))
    nhw = n * hw
    return jnp.concatenate(
        [wp[:nhw], wp[1 : nhw + 1], wp[w : nhw + w], wp[w + 1 : nhw + w + 1]],
        axis=1,
    )


def _post_tc(sc_out, r, c):
    rb = 8

    def body(x_ref, o_ref):
        x = x_ref[...].reshape(rb, 49, c)
        o_ref[...] = jnp.transpose(x, (0, 2, 1))

    return pl.pallas_call(
        body,
        grid=(r // rb,),
        in_specs=[pl.BlockSpec((rb * 49, c), lambda i: (i, 0))],
        out_specs=pl.BlockSpec((rb, c, 49), lambda i: (i, 0, 0)),
        out_shape=jax.ShapeDtypeStruct((r, c, 49), jnp.float32),
    )(sc_out)


def kernel(input, rois):
    n, c, h, w = input.shape
    r = rois.shape[0]
    quad = _build_quad_tc(input)
    idx, wgt = _tap_tables(rois, n, h, w)
    rp = 1024
    n_bins = rp * 49
    idx = jnp.pad(idx, ((0, n_bins - r * 49), (0, 0))).reshape(-1)
    wgt = jnp.pad(wgt, ((0, n_bins - r * 49), (0, 0))).reshape(-1)
    out = _sc_call(n_bins, c, 16)(quad, idx, wgt)
    return _post_tc(out, r, c).reshape(r, c, _OUT_H, _OUT_W)
